# Initial kernel scaffold; baseline (speedup 1.0000x reference)
import jax
import jax.numpy as jnp
from jax.experimental import pallas as pl


def kernel(x1, x2, x3, edge_index1, edge_index2, edge_index3, batch1, batch2, batch3, W1, wih1, whh1, bih1, bhh1, W2, wih2, whh2, bih2, bhh2, W3, wih3, whh3, bih3, bhh3, fc1_w, fc1_b, fc2_w, fc2_b, fc3_w, fc3_b):
    raise NotImplementedError("write your pallas kernel here")


# speedup vs baseline: 1.0097x; 1.0097x over previous
import jax
import jax.numpy as jnp
from jax.experimental import pallas as pl

N_G = 512


def _gated_conv(h, src, dst, W, wih, whh, bih, bhh):
    for i in range(W.shape[0]):
        msg = h @ W[i]
        m = jax.ops.segment_sum(msg[src], dst, num_segments=h.shape[0])
        gi = m @ wih.T + bih
        gh = h @ whh.T + bhh
        i_r, i_z, i_n = jnp.split(gi, 3, axis=1)
        h_r, h_z, h_n = jnp.split(gh, 3, axis=1)
        r = jax.nn.sigmoid(i_r + h_r)
        z = jax.nn.sigmoid(i_z + h_z)
        n = jnp.tanh(i_n + r * h_n)
        h = (1.0 - z) * n + z * h
    return h


def _pool(h, batch):
    s = jax.ops.segment_sum(h, batch, num_segments=N_G)
    c = jax.ops.segment_sum(jnp.ones((h.shape[0],), h.dtype), batch, num_segments=N_G)
    return s / jnp.clip(c, 1.0)[:, None]


def _mlp_kernel(x_ref, w1_ref, b1_ref, w2_ref, b2_ref, w3_ref, b3_ref, o_ref):
    x = x_ref[...]
    y = jax.nn.relu(jnp.dot(x, w1_ref[...].T, preferred_element_type=jnp.float32) + b1_ref[...])
    y = jax.nn.relu(jnp.dot(y, w2_ref[...].T, preferred_element_type=jnp.float32) + b2_ref[...])
    o_ref[...] = jnp.dot(y, w3_ref[...].T, preferred_element_type=jnp.float32) + b3_ref[...]


def kernel(x1, x2, x3, edge_index1, edge_index2, edge_index3, batch1, batch2, batch3,
           W1, wih1, whh1, bih1, bhh1, W2, wih2, whh2, bih2, bhh2, W3, wih3, whh3, bih3, bhh3,
           fc1_w, fc1_b, fc2_w, fc2_b, fc3_w, fc3_b):
    h1 = _pool(jax.nn.relu(_gated_conv(x1, edge_index1[0], edge_index1[1], W1, wih1, whh1, bih1, bhh1)), batch1)
    h2 = _pool(jax.nn.relu(_gated_conv(x2, edge_index2[0], edge_index2[1], W2, wih2, whh2, bih2, bhh2)), batch2)
    h3 = _pool(jax.nn.relu(_gated_conv(x3, edge_index3[0], edge_index3[1], W3, wih3, whh3, bih3, bhh3)), batch3)
    xx = h1 * h2 * h3
    x = jnp.concatenate([h1, h2, h3, xx], axis=1)
    w3p = jnp.zeros((8, fc3_w.shape[1]), fc3_w.dtype).at[:3].set(fc3_w)
    b3p = jnp.zeros((8,), fc3_b.dtype).at[:3].set(fc3_b)
    out = pl.pallas_call(
        _mlp_kernel,
        out_shape=jax.ShapeDtypeStruct((N_G, 8), jnp.float32),
    )(x, fc1_w, fc1_b, fc2_w, fc2_b, w3p, b3p)
    return out[:, :3]


# speedup vs baseline: 2.8428x; 2.8154x over previous
import functools

import jax
import jax.numpy as jnp
from jax import lax
from jax.experimental import pallas as pl
from jax.experimental.pallas import tpu as pltpu
from jax.experimental.pallas import tpu_sc as plsc

N_G = 512
N = 50000
D = 96
DP = 128
EBLK = 128
CHUNK = 8448
CPC = 3
LAST_FULL = 60
LAST_TAIL = 80
ACC_ROWS = 8576
DUMP = CHUNK
NB = 2000
NGRID = N // NB


def _segsum_body(msg_hbm, srcs_hbm, dsts_hbm, offs_hbm, out_hbm,
                 offs_v, src_v, dst_v, dloc_v, rows_v, zrow_v, acc_sh, gsem):
    core = lax.axis_index("c")
    tile = lax.axis_index("s")
    iota = lax.broadcasted_iota(jnp.int32, (16,), 0)

    pltpu.sync_copy(offs_hbm, offs_v)

    def zinit(j, _):
        i = j // (DP // 16)
        k = j % (DP // 16)
        zrow_v[i, pl.ds(k * 16, 16)] = jnp.zeros((16,), jnp.float32)
        return 0
    lax.fori_loop(0, EBLK * (DP // 16), zinit, 0)

    nzb = ACC_ROWS // EBLK
    for r in range(CPC):
        chunk = core * CPC + r
        lo = chunk * CHUNK
        is_last = (core == 1) if r == CPC - 1 else jnp.bool_(False)

        def zb(k, _):
            b = tile + k * 16
            pltpu.sync_copy(zrow_v, acc_sh.at[pl.ds(b * EBLK, EBLK)])
            return 0
        lax.fori_loop(0, (nzb - 1 - tile) // 16 + 1, zb, 0)
        plsc.subcore_barrier()

        offs_vec = offs_v[...]
        start = jnp.where(core == 0, offs_vec[r], offs_vec[r + CPC])
        end = jnp.where(core == 0, offs_vec[r + 1], offs_vec[r + CPC + 1])
        cnt = end - start
        ts = start + (cnt * tile) // 16
        te = start + (cnt * (tile + 1)) // 16
        base0 = (ts // 8) * 8
        nsteps = (te - base0 + EBLK - 1) // EBLK

        def step(j, _):
            base = base0 + j * EBLK
            pltpu.sync_copy(srcs_hbm.at[pl.ds(base, EBLK)], src_v)
            pltpu.sync_copy(dsts_hbm.at[pl.ds(base, EBLK)], dst_v)
            for g in range(EBLK // 16):
                eid = base + g * 16 + iota
                dv = dst_v[pl.ds(g * 16, 16)]
                valid = (eid >= ts) & (eid < te)
                dloc_v[pl.ds(g * 16, 16)] = jnp.where(valid, dv - lo, DUMP)
            pltpu.async_copy(msg_hbm.at[src_v], rows_v, gsem).wait()
            pltpu.sync_copy(rows_v, acc_sh.at[dloc_v], add=True)
            return 0
        lax.fori_loop(0, nsteps, step, 0)
        plsc.subcore_barrier()

        nwb = jnp.where(is_last, LAST_FULL, CHUNK // EBLK)

        def wb(k, _):
            b = tile + k * 16
            pltpu.sync_copy(acc_sh.at[pl.ds(b * EBLK, EBLK)], rows_v)
            pltpu.sync_copy(rows_v, out_hbm.at[pl.ds(lo + b * EBLK, EBLK)])
            return 0
        lax.fori_loop(0, (nwb - 1 - tile) // 16 + 1, wb, 0)

        @pl.when(jnp.logical_and(is_last, tile == 0))
        def _():
            pltpu.sync_copy(acc_sh.at[pl.ds(LAST_FULL * EBLK, LAST_TAIL)],
                            rows_v.at[pl.ds(0, LAST_TAIL)])
            pltpu.sync_copy(rows_v.at[pl.ds(0, LAST_TAIL)],
                            out_hbm.at[pl.ds(lo + LAST_FULL * EBLK, LAST_TAIL)])
        plsc.subcore_barrier()


@functools.lru_cache(maxsize=1)
def _make_segsum():
    return pl.kernel(
        _segsum_body,
        out_type=jax.ShapeDtypeStruct((N, DP), jnp.float32),
        mesh=plsc.VectorSubcoreMesh(core_axis_name="c", subcore_axis_name="s"),
        scratch_types=[
            pltpu.VMEM((16,), jnp.int32),
            pltpu.VMEM((EBLK,), jnp.int32),
            pltpu.VMEM((EBLK,), jnp.int32),
            pltpu.VMEM((EBLK,), jnp.int32),
            pltpu.VMEM((EBLK, DP), jnp.float32),
            pltpu.VMEM((EBLK, DP), jnp.float32),
            pltpu.VMEM_SHARED((ACC_ROWS, DP), jnp.float32),
            pltpu.SemaphoreType.DMA,
        ],
    )


def _segsum(msg, srcs, dsts, offs):
    return _make_segsum()(msg, srcs, dsts, offs)


def _prep_edges(edge_index):
    src = edge_index[0]
    dst = edge_index[1]
    e = src.shape[0]
    dsts, srcs = lax.sort_key_val(dst, src)
    bounds = jnp.arange(1, 2 * CPC, dtype=jnp.int32) * CHUNK
    mids = jnp.searchsorted(dsts, bounds).astype(jnp.int32)
    offs = jnp.concatenate([jnp.zeros((1,), jnp.int32), mids,
                            jnp.full((1,), e, jnp.int32)])
    offs16 = jnp.zeros((16,), jnp.int32).at[:2 * CPC + 1].set(offs)
    pad = jnp.zeros((EBLK,), jnp.int32)
    return jnp.concatenate([srcs, pad]), jnp.concatenate([dsts, pad]), offs16


def _mm_body(x_ref, w_ref, o_ref):
    o_ref[...] = jnp.dot(x_ref[...], w_ref[...], preferred_element_type=jnp.float32)


def _mm(x, w):
    return pl.pallas_call(
        _mm_body,
        grid=(NGRID,),
        in_specs=[pl.BlockSpec((NB, D), lambda i: (i, 0)),
                  pl.BlockSpec((D, DP), lambda i: (0, 0))],
        out_specs=pl.BlockSpec((NB, DP), lambda i: (i, 0)),
        out_shape=jax.ShapeDtypeStruct((N, DP), jnp.float32),
    )(x, w)


def _gru_body(m_ref, h_ref, wr_ref, wz_ref, wn_ref, ur_ref, uz_ref, un_ref,
              brz_ref, bn_i_ref, bn_h_ref, wnext_ref, h_out, msg_out):
    m = m_ref[...]
    h = h_ref[...]
    dot = lambda a, b: jnp.dot(a, b, preferred_element_type=jnp.float32)
    r = jax.nn.sigmoid(dot(m, wr_ref[...]) + dot(h, ur_ref[...]) + brz_ref[0:1, :])
    z = jax.nn.sigmoid(dot(m, wz_ref[...]) + dot(h, uz_ref[...]) + brz_ref[1:2, :])
    nn = jnp.tanh(dot(m, wn_ref[...]) + bn_i_ref[...] +
                  r * (dot(h, un_ref[...]) + bn_h_ref[...]))
    hn = (1.0 - z) * nn + z * h
    h_out[...] = hn
    msg_out[...] = dot(hn, wnext_ref[...])


def _gru_final_body(m_ref, h_ref, wr_ref, wz_ref, wn_ref, ur_ref, uz_ref, un_ref,
                    brz_ref, bn_i_ref, bn_h_ref, h_out):
    m = m_ref[...]
    h = h_ref[...]
    dot = lambda a, b: jnp.dot(a, b, preferred_element_type=jnp.float32)
    r = jax.nn.sigmoid(dot(m, wr_ref[...]) + dot(h, ur_ref[...]) + brz_ref[0:1, :])
    z = jax.nn.sigmoid(dot(m, wz_ref[...]) + dot(h, uz_ref[...]) + brz_ref[1:2, :])
    nn = jnp.tanh(dot(m, wn_ref[...]) + bn_i_ref[...] +
                  r * (dot(h, un_ref[...]) + bn_h_ref[...]))
    h_out[...] = jax.nn.relu((1.0 - z) * nn + z * h)


def _gru(m, h, gw, wnext):
    blk = lambda: pl.BlockSpec((NB, D), lambda i: (i, 0))
    mblk = lambda: pl.BlockSpec((NB, DP), lambda i: (i, 0))
    wspec = lambda: pl.BlockSpec((D, D), lambda i: (0, 0))
    mwspec = lambda: pl.BlockSpec((DP, D), lambda i: (0, 0))
    if wnext is None:
        return pl.pallas_call(
            _gru_final_body,
            grid=(NGRID,),
            in_specs=[mblk(), blk()] + [mwspec()] * 3 + [wspec()] * 3 +
                     [pl.BlockSpec((2, D), lambda i: (0, 0)),
                      pl.BlockSpec((1, D), lambda i: (0, 0)),
                      pl.BlockSpec((1, D), lambda i: (0, 0))],
            out_specs=blk(),
            out_shape=jax.ShapeDtypeStruct((N, D), jnp.float32),
        )(m, h, *gw)
    return pl.pallas_call(
        _gru_body,
        grid=(NGRID,),
        in_specs=[mblk(), blk()] + [mwspec()] * 3 + [wspec()] * 3 +
                 [pl.BlockSpec((2, D), lambda i: (0, 0)),
                  pl.BlockSpec((1, D), lambda i: (0, 0)),
                  pl.BlockSpec((1, D), lambda i: (0, 0)),
                  pl.BlockSpec((D, DP), lambda i: (0, 0))],
        out_specs=[blk(), mblk()],
        out_shape=[jax.ShapeDtypeStruct((N, D), jnp.float32),
                   jax.ShapeDtypeStruct((N, DP), jnp.float32)],
    )(m, h, *gw, wnext)


def _pool_body(b_ref, h_ref, s_ref, c_ref):
    pi = pl.program_id(0)

    @pl.when(pi == 0)
    def _():
        s_ref[...] = jnp.zeros_like(s_ref)
        c_ref[...] = jnp.zeros_like(c_ref)

    bcol = b_ref[0]
    oh = (bcol == lax.broadcasted_iota(jnp.int32, (NB, N_G), 1)).astype(jnp.float32)
    h = h_ref[...]
    s_ref[...] += lax.dot_general(oh, h, (((0,), (0,)), ((), ())),
                                  preferred_element_type=jnp.float32)
    c_ref[...] += lax.dot_general(oh, jnp.ones((NB, 8), jnp.float32),
                                  (((0,), (0,)), ((), ())),
                                  preferred_element_type=jnp.float32)


def _pool(h, batch3d):
    return pl.pallas_call(
        _pool_body,
        grid=(NGRID,),
        in_specs=[pl.BlockSpec((1, NB, 1), lambda i: (i, 0, 0)),
                  pl.BlockSpec((NB, D), lambda i: (i, 0))],
        out_specs=[pl.BlockSpec((N_G, D), lambda i: (0, 0)),
                   pl.BlockSpec((N_G, 8), lambda i: (0, 0))],
        out_shape=[jax.ShapeDtypeStruct((N_G, D), jnp.float32),
                   jax.ShapeDtypeStruct((N_G, 8), jnp.float32)],
    )(batch3d, h)


def _mlp_body(s1_ref, c1_ref, s2_ref, c2_ref, s3_ref, c3_ref,
              a1_ref, a2_ref, a3_ref, a4_ref, b1_ref, w2_ref, b2_ref,
              w3_ref, b3_ref, o_ref):
    h1 = s1_ref[...] / jnp.maximum(c1_ref[:, 0:1], 1.0)
    h2 = s2_ref[...] / jnp.maximum(c2_ref[:, 0:1], 1.0)
    h3 = s3_ref[...] / jnp.maximum(c3_ref[:, 0:1], 1.0)
    xx = h1 * h2 * h3
    dot = lambda a, b: jnp.dot(a, b, preferred_element_type=jnp.float32)
    y = jax.nn.relu(dot(h1, a1_ref[...]) + dot(h2, a2_ref[...]) +
                    dot(h3, a3_ref[...]) + dot(xx, a4_ref[...]) + b1_ref[...])
    y = jax.nn.relu(dot(y, w2_ref[...]) + b2_ref[...])
    o_ref[...] = dot(y, w3_ref[...]) + b3_ref[...]


def _branch(x, edge_index, W, wih, whh, bih, bhh):
    srcs, dsts, offs = _prep_edges(edge_index)
    zpad = jnp.zeros((DP - D, D), jnp.float32)
    wr = jnp.concatenate([wih[0:D].T, zpad])
    wz = jnp.concatenate([wih[D:2 * D].T, zpad])
    wn = jnp.concatenate([wih[2 * D:].T, zpad])
    ur, uz, un = whh[0:D].T, whh[D:2 * D].T, whh[2 * D:].T
    brz = jnp.stack([bih[0:D] + bhh[0:D], bih[D:2 * D] + bhh[D:2 * D]])
    bn_i = bih[2 * D:].reshape(1, D)
    bn_h = bhh[2 * D:].reshape(1, D)
    gw = (wr, wz, wn, ur, uz, un, brz, bn_i, bn_h)

    L = W.shape[0]
    cpad = jnp.zeros((D, DP - D), jnp.float32)
    Wp = [jnp.concatenate([W[i], cpad], axis=1) for i in range(L)]
    h = x
    msg = _mm(x, Wp[0])
    for i in range(L):
        m = _segsum(msg, srcs, dsts, offs)
        if i + 1 < L:
            h, msg = _gru(m, h, gw, Wp[i + 1])
        else:
            h = _gru(m, h, gw, None)
    return h


def kernel(x1, x2, x3, edge_index1, edge_index2, edge_index3, batch1, batch2, batch3,
           W1, wih1, whh1, bih1, bhh1, W2, wih2, whh2, bih2, bhh2, W3, wih3, whh3, bih3, bhh3,
           fc1_w, fc1_b, fc2_w, fc2_b, fc3_w, fc3_b):
    hf1 = _branch(x1, edge_index1, W1, wih1, whh1, bih1, bhh1)
    hf2 = _branch(x2, edge_index2, W2, wih2, whh2, bih2, bhh2)
    hf3 = _branch(x3, edge_index3, W3, wih3, whh3, bih3, bhh3)

    s1, c1 = _pool(hf1, batch1.reshape(NGRID, NB, 1))
    s2, c2 = _pool(hf2, batch2.reshape(NGRID, NB, 1))
    s3, c3 = _pool(hf3, batch3.reshape(NGRID, NB, 1))

    fc_dim = fc1_w.shape[1]
    w1t = fc1_w.T
    a1, a2, a3, a4 = w1t[0:D], w1t[D:2 * D], w1t[2 * D:3 * D], w1t[3 * D:]
    b1 = fc1_b.reshape(1, -1)
    w2t = fc2_w.T
    b2 = fc2_b.reshape(1, -1)
    w3t = jnp.zeros((fc_dim, 8), fc3_w.dtype).at[:, :3].set(fc3_w.T)
    b3 = jnp.zeros((1, 8), fc3_b.dtype).at[0, :3].set(fc3_b)

    out = pl.pallas_call(
        _mlp_body,
        out_shape=jax.ShapeDtypeStruct((N_G, 8), jnp.float32),
    )(s1, c1, s2, c2, s3, c3, a1, a2, a3, a4, b1, w2t, b2, w3t, b3)
    return out[:, :3]


# speedup vs baseline: 3.8501x; 1.3543x over previous
import functools

import jax
import jax.numpy as jnp
from jax import lax
from jax.experimental import pallas as pl
from jax.experimental.pallas import tpu as pltpu
from jax.experimental.pallas import tpu_sc as plsc

N_G = 512
N = 50000
D = 96
DP = 128
EBLK = 128
IDXB = 8
ZROWS = 64
CHUNK = 8448
CPC = 3
LAST_FULL = 60
LAST_TAIL = 80
ACC_ROWS = 8576
DUMP = CHUNK
NB = 2000
NGRID = N // NB


def _segsum_body(msg_hbm, srcs_hbm, dsts_hbm, offs_hbm, out_hbm,
                 offs_v, srcb_v, dstb_v, dloc_v, rows0_v, rows1_v, zrow_v,
                 acc_sh, gsem0, gsem1, ssem0, ssem1):
    core = lax.axis_index("c")
    tile = lax.axis_index("s")
    iota = lax.broadcasted_iota(jnp.int32, (16,), 0)

    pltpu.sync_copy(offs_hbm, offs_v)
    offs_vec = offs_v[...]

    def zinit(j, _):
        i = j // (DP // 16)
        k = j % (DP // 16)
        zrow_v[i, pl.ds(k * 16, 16)] = jnp.zeros((16,), jnp.float32)
        return 0
    lax.fori_loop(0, ZROWS * (DP // 16), zinit, 0)

    def gather_desc(s, rows_ref, sem):
        return pltpu.make_async_copy(
            msg_hbm.at[srcb_v.at[pl.ds(s * EBLK, EBLK)]], rows_ref, sem)

    def scat_desc(s, rows_ref, sem):
        return pltpu.make_async_copy(rows_ref, acc_sh.at[dloc_v.at[s]], sem)

    nzb = ACC_ROWS // ZROWS
    for r in range(CPC):
        chunk = core * CPC + r
        lo = chunk * CHUNK
        is_last = (core == 1) if r == CPC - 1 else jnp.bool_(False)

        def zb(k, _):
            b = tile + k * 16
            pltpu.sync_copy(zrow_v, acc_sh.at[pl.ds(b * ZROWS, ZROWS)])
            return 0
        lax.fori_loop(0, (nzb - 1 - tile) // 16 + 1, zb, 0)
        plsc.subcore_barrier()

        start = jnp.where(core == 0, offs_vec[r], offs_vec[r + CPC])
        end = jnp.where(core == 0, offs_vec[r + 1], offs_vec[r + CPC + 1])
        cnt = end - start
        ts = start + (cnt * tile) // 16
        te = start + (cnt * (tile + 1)) // 16
        base0 = (ts // 8) * 8
        nsteps = (te - base0 + EBLK - 1) // EBLK
        nblocks = (nsteps + IDXB - 1) // IDXB

        def block(kb, _):
            jbase = kb * IDXB
            ebase = base0 + jbase * EBLK

            @pl.when(kb >= 1)
            def _():
                scat_desc(0, rows0_v, ssem0).wait()
                scat_desc(0, rows1_v, ssem1).wait()

            pltpu.sync_copy(srcs_hbm.at[pl.ds(ebase, IDXB * EBLK)], srcb_v)
            pltpu.sync_copy(dsts_hbm.at[pl.ds(ebase, IDXB * EBLK)], dstb_v)

            def dl(i, _):
                eid = ebase + i * 16 + iota
                dv = dstb_v[pl.ds(i * 16, 16)]
                valid = (eid >= ts) & (eid < te)
                si = i // (EBLK // 16)
                gi = i % (EBLK // 16)
                dloc_v[si, pl.ds(gi * 16, 16)] = jnp.where(valid, dv - lo, DUMP)
                return 0
            lax.fori_loop(0, IDXB * EBLK // 16, dl, 0)

            for u in range(IDXB // 2):
                j0 = jbase + 2 * u
                j1 = j0 + 1
                s0, s1 = 2 * u, 2 * u + 1

                @pl.when(j0 < nsteps)
                def _(u=u, s0=s0):
                    if u >= 1:
                        scat_desc(s0, rows0_v, ssem0).wait()
                    pltpu.async_copy(
                        msg_hbm.at[srcb_v.at[pl.ds(s0 * EBLK, EBLK)]],
                        rows0_v, gsem0)

                @pl.when(j1 < nsteps)
                def _(u=u, s1=s1):
                    if u >= 1:
                        scat_desc(s1, rows1_v, ssem1).wait()
                    pltpu.async_copy(
                        msg_hbm.at[srcb_v.at[pl.ds(s1 * EBLK, EBLK)]],
                        rows1_v, gsem1)

                @pl.when(j0 < nsteps)
                def _(s0=s0):
                    gather_desc(s0, rows0_v, gsem0).wait()
                    pltpu.async_copy(rows0_v, acc_sh.at[dloc_v.at[s0]],
                                     ssem0, add=True)

                @pl.when(j1 < nsteps)
                def _(s1=s1):
                    gather_desc(s1, rows1_v, gsem1).wait()
                    pltpu.async_copy(rows1_v, acc_sh.at[dloc_v.at[s1]],
                                     ssem1, add=True)
            return 0
        lax.fori_loop(0, nblocks, block, 0)

        @pl.when(nsteps >= 1)
        def _():
            scat_desc(0, rows0_v, ssem0).wait()

        @pl.when(nsteps >= 2)
        def _():
            scat_desc(0, rows1_v, ssem1).wait()
        plsc.subcore_barrier()

        nwb = jnp.where(is_last, LAST_FULL, CHUNK // EBLK)

        def wb(k, _):
            b = tile + k * 16
            pltpu.sync_copy(acc_sh.at[pl.ds(b * EBLK, EBLK)], rows0_v)
            pltpu.sync_copy(rows0_v, out_hbm.at[pl.ds(lo + b * EBLK, EBLK)])
            return 0
        lax.fori_loop(0, (nwb - 1 - tile) // 16 + 1, wb, 0)

        @pl.when(jnp.logical_and(is_last, tile == 0))
        def _():
            pltpu.sync_copy(acc_sh.at[pl.ds(LAST_FULL * EBLK, LAST_TAIL)],
                            rows0_v.at[pl.ds(0, LAST_TAIL)])
            pltpu.sync_copy(rows0_v.at[pl.ds(0, LAST_TAIL)],
                            out_hbm.at[pl.ds(lo + LAST_FULL * EBLK, LAST_TAIL)])
        plsc.subcore_barrier()


@functools.lru_cache(maxsize=1)
def _make_segsum():
    return pl.kernel(
        _segsum_body,
        out_type=jax.ShapeDtypeStruct((N, DP), jnp.float32),
        mesh=plsc.VectorSubcoreMesh(core_axis_name="c", subcore_axis_name="s"),
        scratch_types=[
            pltpu.VMEM((16,), jnp.int32),
            pltpu.VMEM((IDXB * EBLK,), jnp.int32),
            pltpu.VMEM((IDXB * EBLK,), jnp.int32),
            pltpu.VMEM((IDXB, EBLK), jnp.int32),
            pltpu.VMEM((EBLK, DP), jnp.float32),
            pltpu.VMEM((EBLK, DP), jnp.float32),
            pltpu.VMEM((ZROWS, DP), jnp.float32),
            pltpu.VMEM_SHARED((ACC_ROWS, DP), jnp.float32),
            pltpu.SemaphoreType.DMA,
            pltpu.SemaphoreType.DMA,
            pltpu.SemaphoreType.DMA,
            pltpu.SemaphoreType.DMA,
        ],
    )


def _segsum(msg, srcs, dsts, offs):
    return _make_segsum()(msg, srcs, dsts, offs)


def _prep_edges(edge_index):
    src = edge_index[0]
    dst = edge_index[1]
    e = src.shape[0]
    dsts, srcs = lax.sort_key_val(dst, src)
    bounds = jnp.arange(1, 2 * CPC, dtype=jnp.int32) * CHUNK
    mids = jnp.searchsorted(dsts, bounds).astype(jnp.int32)
    offs = jnp.concatenate([jnp.zeros((1,), jnp.int32), mids,
                            jnp.full((1,), e, jnp.int32)])
    offs16 = jnp.zeros((16,), jnp.int32).at[:2 * CPC + 1].set(offs)
    pad = jnp.zeros((IDXB * EBLK,), jnp.int32)
    return jnp.concatenate([srcs, pad]), jnp.concatenate([dsts, pad]), offs16


def _mm_body(x_ref, w_ref, o_ref):
    o_ref[...] = jnp.dot(x_ref[...], w_ref[...], preferred_element_type=jnp.float32)


def _mm(x, w):
    return pl.pallas_call(
        _mm_body,
        grid=(NGRID,),
        in_specs=[pl.BlockSpec((NB, D), lambda i: (i, 0)),
                  pl.BlockSpec((D, DP), lambda i: (0, 0))],
        out_specs=pl.BlockSpec((NB, DP), lambda i: (i, 0)),
        out_shape=jax.ShapeDtypeStruct((N, DP), jnp.float32),
    )(x, w)


def _gru_body(m_ref, h_ref, wr_ref, wz_ref, wn_ref, ur_ref, uz_ref, un_ref,
              brz_ref, bn_i_ref, bn_h_ref, wnext_ref, h_out, msg_out):
    m = m_ref[...]
    h = h_ref[...]
    dot = lambda a, b: jnp.dot(a, b, preferred_element_type=jnp.float32)
    r = jax.nn.sigmoid(dot(m, wr_ref[...]) + dot(h, ur_ref[...]) + brz_ref[0:1, :])
    z = jax.nn.sigmoid(dot(m, wz_ref[...]) + dot(h, uz_ref[...]) + brz_ref[1:2, :])
    nn = jnp.tanh(dot(m, wn_ref[...]) + bn_i_ref[...] +
                  r * (dot(h, un_ref[...]) + bn_h_ref[...]))
    hn = (1.0 - z) * nn + z * h
    h_out[...] = hn
    msg_out[...] = dot(hn, wnext_ref[...])


def _gru_final_body(m_ref, h_ref, wr_ref, wz_ref, wn_ref, ur_ref, uz_ref, un_ref,
                    brz_ref, bn_i_ref, bn_h_ref, h_out):
    m = m_ref[...]
    h = h_ref[...]
    dot = lambda a, b: jnp.dot(a, b, preferred_element_type=jnp.float32)
    r = jax.nn.sigmoid(dot(m, wr_ref[...]) + dot(h, ur_ref[...]) + brz_ref[0:1, :])
    z = jax.nn.sigmoid(dot(m, wz_ref[...]) + dot(h, uz_ref[...]) + brz_ref[1:2, :])
    nn = jnp.tanh(dot(m, wn_ref[...]) + bn_i_ref[...] +
                  r * (dot(h, un_ref[...]) + bn_h_ref[...]))
    h_out[...] = jax.nn.relu((1.0 - z) * nn + z * h)


def _gru(m, h, gw, wnext):
    blk = lambda: pl.BlockSpec((NB, D), lambda i: (i, 0))
    mblk = lambda: pl.BlockSpec((NB, DP), lambda i: (i, 0))
    wspec = lambda: pl.BlockSpec((D, D), lambda i: (0, 0))
    mwspec = lambda: pl.BlockSpec((DP, D), lambda i: (0, 0))
    if wnext is None:
        return pl.pallas_call(
            _gru_final_body,
            grid=(NGRID,),
            in_specs=[mblk(), blk()] + [mwspec()] * 3 + [wspec()] * 3 +
                     [pl.BlockSpec((2, D), lambda i: (0, 0)),
                      pl.BlockSpec((1, D), lambda i: (0, 0)),
                      pl.BlockSpec((1, D), lambda i: (0, 0))],
            out_specs=blk(),
            out_shape=jax.ShapeDtypeStruct((N, D), jnp.float32),
        )(m, h, *gw)
    return pl.pallas_call(
        _gru_body,
        grid=(NGRID,),
        in_specs=[mblk(), blk()] + [mwspec()] * 3 + [wspec()] * 3 +
                 [pl.BlockSpec((2, D), lambda i: (0, 0)),
                  pl.BlockSpec((1, D), lambda i: (0, 0)),
                  pl.BlockSpec((1, D), lambda i: (0, 0)),
                  pl.BlockSpec((D, DP), lambda i: (0, 0))],
        out_specs=[blk(), mblk()],
        out_shape=[jax.ShapeDtypeStruct((N, D), jnp.float32),
                   jax.ShapeDtypeStruct((N, DP), jnp.float32)],
    )(m, h, *gw, wnext)


def _pool_body(b_ref, h_ref, s_ref, c_ref):
    pi = pl.program_id(0)

    @pl.when(pi == 0)
    def _():
        s_ref[...] = jnp.zeros_like(s_ref)
        c_ref[...] = jnp.zeros_like(c_ref)

    bcol = b_ref[0]
    oh = (bcol == lax.broadcasted_iota(jnp.int32, (NB, N_G), 1)).astype(jnp.float32)
    h = h_ref[...]
    s_ref[...] += lax.dot_general(oh, h, (((0,), (0,)), ((), ())),
                                  preferred_element_type=jnp.float32)
    c_ref[...] += lax.dot_general(oh, jnp.ones((NB, 8), jnp.float32),
                                  (((0,), (0,)), ((), ())),
                                  preferred_element_type=jnp.float32)


def _pool(h, batch3d):
    return pl.pallas_call(
        _pool_body,
        grid=(NGRID,),
        in_specs=[pl.BlockSpec((1, NB, 1), lambda i: (i, 0, 0)),
                  pl.BlockSpec((NB, D), lambda i: (i, 0))],
        out_specs=[pl.BlockSpec((N_G, D), lambda i: (0, 0)),
                   pl.BlockSpec((N_G, 8), lambda i: (0, 0))],
        out_shape=[jax.ShapeDtypeStruct((N_G, D), jnp.float32),
                   jax.ShapeDtypeStruct((N_G, 8), jnp.float32)],
    )(batch3d, h)


def _mlp_body(s1_ref, c1_ref, s2_ref, c2_ref, s3_ref, c3_ref,
              a1_ref, a2_ref, a3_ref, a4_ref, b1_ref, w2_ref, b2_ref,
              w3_ref, b3_ref, o_ref):
    h1 = s1_ref[...] / jnp.maximum(c1_ref[:, 0:1], 1.0)
    h2 = s2_ref[...] / jnp.maximum(c2_ref[:, 0:1], 1.0)
    h3 = s3_ref[...] / jnp.maximum(c3_ref[:, 0:1], 1.0)
    xx = h1 * h2 * h3
    dot = lambda a, b: jnp.dot(a, b, preferred_element_type=jnp.float32)
    y = jax.nn.relu(dot(h1, a1_ref[...]) + dot(h2, a2_ref[...]) +
                    dot(h3, a3_ref[...]) + dot(xx, a4_ref[...]) + b1_ref[...])
    y = jax.nn.relu(dot(y, w2_ref[...]) + b2_ref[...])
    o_ref[...] = dot(y, w3_ref[...]) + b3_ref[...]


def _branch(x, edge_index, W, wih, whh, bih, bhh):
    srcs, dsts, offs = _prep_edges(edge_index)
    zpad = jnp.zeros((DP - D, D), jnp.float32)
    wr = jnp.concatenate([wih[0:D].T, zpad])
    wz = jnp.concatenate([wih[D:2 * D].T, zpad])
    wn = jnp.concatenate([wih[2 * D:].T, zpad])
    ur, uz, un = whh[0:D].T, whh[D:2 * D].T, whh[2 * D:].T
    brz = jnp.stack([bih[0:D] + bhh[0:D], bih[D:2 * D] + bhh[D:2 * D]])
    bn_i = bih[2 * D:].reshape(1, D)
    bn_h = bhh[2 * D:].reshape(1, D)
    gw = (wr, wz, wn, ur, uz, un, brz, bn_i, bn_h)

    L = W.shape[0]
    cpad = jnp.zeros((D, DP - D), jnp.float32)
    Wp = [jnp.concatenate([W[i], cpad], axis=1) for i in range(L)]
    h = x
    msg = _mm(x, Wp[0])
    for i in range(L):
        m = _segsum(msg, srcs, dsts, offs)
        if i + 1 < L:
            h, msg = _gru(m, h, gw, Wp[i + 1])
        else:
            h = _gru(m, h, gw, None)
    return h


def kernel(x1, x2, x3, edge_index1, edge_index2, edge_index3, batch1, batch2, batch3,
           W1, wih1, whh1, bih1, bhh1, W2, wih2, whh2, bih2, bhh2, W3, wih3, whh3, bih3, bhh3,
           fc1_w, fc1_b, fc2_w, fc2_b, fc3_w, fc3_b):
    hf1 = _branch(x1, edge_index1, W1, wih1, whh1, bih1, bhh1)
    hf2 = _branch(x2, edge_index2, W2, wih2, whh2, bih2, bhh2)
    hf3 = _branch(x3, edge_index3, W3, wih3, whh3, bih3, bhh3)

    s1, c1 = _pool(hf1, batch1.reshape(NGRID, NB, 1))
    s2, c2 = _pool(hf2, batch2.reshape(NGRID, NB, 1))
    s3, c3 = _pool(hf3, batch3.reshape(NGRID, NB, 1))

    fc_dim = fc1_w.shape[1]
    w1t = fc1_w.T
    a1, a2, a3, a4 = w1t[0:D], w1t[D:2 * D], w1t[2 * D:3 * D], w1t[3 * D:]
    b1 = fc1_b.reshape(1, -1)
    w2t = fc2_w.T
    b2 = fc2_b.reshape(1, -1)
    w3t = jnp.zeros((fc_dim, 8), fc3_w.dtype).at[:, :3].set(fc3_w.T)
    b3 = jnp.zeros((1, 8), fc3_b.dtype).at[0, :3].set(fc3_b)

    out = pl.pallas_call(
        _mlp_body,
        out_shape=jax.ShapeDtypeStruct((N_G, 8), jnp.float32),
    )(s1, c1, s2, c2, s3, c3, a1, a2, a3, a4, b1, w2t, b2, w3t, b3)
    return out[:, :3]


# speedup vs baseline: 4.0184x; 1.0437x over previous
import functools

import jax
import jax.numpy as jnp
from jax import lax
from jax.experimental import pallas as pl
from jax.experimental.pallas import tpu as pltpu
from jax.experimental.pallas import tpu_sc as plsc

N_G = 512
N = 50000
D = 96
DP = 128
EBLK = 128
IDXB = 8
ZROWS = 64
CHUNK = 8448
CPC = 3
LAST_FULL = 60
LAST_TAIL = 80
ACC_ROWS = 8576
DUMP = CHUNK
NB = 2000
NGRID = N // NB


def _segsum_body(msg_hbm, srcs_hbm, dsts_hbm, offs_hbm, out_hbm,
                 offs_v, srcb_v, dstb_v, dloc_v, rows0_v, rows1_v, zrow_v,
                 acc_sh, gsem0, gsem1, ssem0, ssem1):
    core = lax.axis_index("c")
    tile = lax.axis_index("s")
    iota = lax.broadcasted_iota(jnp.int32, (16,), 0)

    pltpu.sync_copy(offs_hbm, offs_v)
    offs_vec = offs_v[...]

    def zinit(j, _):
        i = j // (DP // 16)
        k = j % (DP // 16)
        zrow_v[i, pl.ds(k * 16, 16)] = jnp.zeros((16,), jnp.float32)
        return 0
    lax.fori_loop(0, ZROWS * (DP // 16), zinit, 0)

    def gather_desc(s, rows_ref, sem):
        return pltpu.make_async_copy(
            msg_hbm.at[srcb_v.at[pl.ds(s * EBLK, EBLK)]], rows_ref, sem)

    def scat_desc(s, rows_ref, sem):
        return pltpu.make_async_copy(rows_ref, acc_sh.at[dloc_v.at[s]], sem)

    nzb = ACC_ROWS // ZROWS
    for r in range(CPC):
        chunk = core * CPC + r
        lo = chunk * CHUNK
        is_last = (core == 1) if r == CPC - 1 else jnp.bool_(False)

        def zb(k, _):
            b = tile + k * 16
            pltpu.sync_copy(zrow_v, acc_sh.at[pl.ds(b * ZROWS, ZROWS)])
            return 0
        lax.fori_loop(0, (nzb - 1 - tile) // 16 + 1, zb, 0)
        plsc.subcore_barrier()

        start = jnp.where(core == 0, offs_vec[r], offs_vec[r + CPC])
        end = jnp.where(core == 0, offs_vec[r + 1], offs_vec[r + CPC + 1])
        cnt = end - start
        ts = start + (cnt * tile) // 16
        te = start + (cnt * (tile + 1)) // 16
        base0 = (ts // 8) * 8
        nsteps = (te - base0 + EBLK - 1) // EBLK
        nblocks = (nsteps + IDXB - 1) // IDXB

        def block(kb, _):
            jbase = kb * IDXB
            ebase = base0 + jbase * EBLK

            pltpu.sync_copy(srcs_hbm.at[pl.ds(ebase, IDXB * EBLK)], srcb_v)
            pltpu.sync_copy(dsts_hbm.at[pl.ds(ebase, IDXB * EBLK)], dstb_v)

            def dl(i, _):
                eid = ebase + i * 16 + iota
                dv = dstb_v[pl.ds(i * 16, 16)]
                valid = (eid >= ts) & (eid < te)
                si = i // (EBLK // 16)
                gi = i % (EBLK // 16)
                dloc_v[si, pl.ds(gi * 16, 16)] = jnp.where(valid, dv - lo, DUMP)
                return 0
            lax.fori_loop(0, IDXB * EBLK // 16, dl, 0)

            for u in range(IDXB // 2):
                j0 = jbase + 2 * u
                j1 = j0 + 1
                s0, s1 = 2 * u, 2 * u + 1

                @pl.when(j0 < nsteps)
                def _(j0=j0, s0=s0):
                    @pl.when(j0 >= 2)
                    def _():
                        scat_desc(s0, rows0_v, ssem0).wait()
                    pltpu.async_copy(
                        msg_hbm.at[srcb_v.at[pl.ds(s0 * EBLK, EBLK)]],
                        rows0_v, gsem0)

                @pl.when(j1 < nsteps)
                def _(j1=j1, s1=s1):
                    @pl.when(j1 >= 2)
                    def _():
                        scat_desc(s1, rows1_v, ssem1).wait()
                    pltpu.async_copy(
                        msg_hbm.at[srcb_v.at[pl.ds(s1 * EBLK, EBLK)]],
                        rows1_v, gsem1)

                @pl.when(j0 < nsteps)
                def _(s0=s0):
                    gather_desc(s0, rows0_v, gsem0).wait()
                    pltpu.async_copy(rows0_v, acc_sh.at[dloc_v.at[s0]],
                                     ssem0, add=True)

                @pl.when(j1 < nsteps)
                def _(s1=s1):
                    gather_desc(s1, rows1_v, gsem1).wait()
                    pltpu.async_copy(rows1_v, acc_sh.at[dloc_v.at[s1]],
                                     ssem1, add=True)
            return 0
        lax.fori_loop(0, nblocks, block, 0)

        @pl.when(nsteps >= 1)
        def _():
            scat_desc(0, rows0_v, ssem0).wait()

        @pl.when(nsteps >= 2)
        def _():
            scat_desc(0, rows1_v, ssem1).wait()
        plsc.subcore_barrier()

        nwb = jnp.where(is_last, LAST_FULL, CHUNK // EBLK)

        def wb(k, _):
            b = tile + k * 16
            pltpu.sync_copy(acc_sh.at[pl.ds(b * EBLK, EBLK)], rows0_v)
            pltpu.sync_copy(rows0_v, out_hbm.at[pl.ds(lo + b * EBLK, EBLK)])
            return 0
        lax.fori_loop(0, (nwb - 1 - tile) // 16 + 1, wb, 0)

        @pl.when(jnp.logical_and(is_last, tile == 0))
        def _():
            pltpu.sync_copy(acc_sh.at[pl.ds(LAST_FULL * EBLK, LAST_TAIL)],
                            rows0_v.at[pl.ds(0, LAST_TAIL)])
            pltpu.sync_copy(rows0_v.at[pl.ds(0, LAST_TAIL)],
                            out_hbm.at[pl.ds(lo + LAST_FULL * EBLK, LAST_TAIL)])
        plsc.subcore_barrier()


@functools.lru_cache(maxsize=1)
def _make_segsum():
    return pl.kernel(
        _segsum_body,
        out_type=jax.ShapeDtypeStruct((N, DP), jnp.float32),
        mesh=plsc.VectorSubcoreMesh(core_axis_name="c", subcore_axis_name="s"),
        scratch_types=[
            pltpu.VMEM((16,), jnp.int32),
            pltpu.VMEM((IDXB * EBLK,), jnp.int32),
            pltpu.VMEM((IDXB * EBLK,), jnp.int32),
            pltpu.VMEM((IDXB, EBLK), jnp.int32),
            pltpu.VMEM((EBLK, DP), jnp.float32),
            pltpu.VMEM((EBLK, DP), jnp.float32),
            pltpu.VMEM((ZROWS, DP), jnp.float32),
            pltpu.VMEM_SHARED((ACC_ROWS, DP), jnp.float32),
            pltpu.SemaphoreType.DMA,
            pltpu.SemaphoreType.DMA,
            pltpu.SemaphoreType.DMA,
            pltpu.SemaphoreType.DMA,
        ],
    )


def _segsum(msg, srcs, dsts, offs):
    return _make_segsum()(msg, srcs, dsts, offs)


def _prep_edges(edge_index):
    src = edge_index[0]
    dst = edge_index[1]
    e = src.shape[0]
    dsts, srcs = lax.sort_key_val(dst, src)
    bounds = jnp.arange(1, 2 * CPC, dtype=jnp.int32) * CHUNK
    mids = jnp.searchsorted(dsts, bounds).astype(jnp.int32)
    offs = jnp.concatenate([jnp.zeros((1,), jnp.int32), mids,
                            jnp.full((1,), e, jnp.int32)])
    offs16 = jnp.zeros((16,), jnp.int32).at[:2 * CPC + 1].set(offs)
    pad = jnp.zeros((IDXB * EBLK,), jnp.int32)
    return jnp.concatenate([srcs, pad]), jnp.concatenate([dsts, pad]), offs16


def _mm_body(x_ref, w_ref, o_ref):
    o_ref[...] = jnp.dot(x_ref[...], w_ref[...], preferred_element_type=jnp.float32)


def _mm(x, w):
    return pl.pallas_call(
        _mm_body,
        grid=(NGRID,),
        in_specs=[pl.BlockSpec((NB, D), lambda i: (i, 0)),
                  pl.BlockSpec((D, DP), lambda i: (0, 0))],
        out_specs=pl.BlockSpec((NB, DP), lambda i: (i, 0)),
        out_shape=jax.ShapeDtypeStruct((N, DP), jnp.float32),
    )(x, w)


def _gru_body(m_ref, h_ref, wr_ref, wz_ref, wn_ref, ur_ref, uz_ref, un_ref,
              brz_ref, bn_i_ref, bn_h_ref, wnext_ref, h_out, msg_out):
    m = m_ref[...]
    h = h_ref[...]
    dot = lambda a, b: jnp.dot(a, b, preferred_element_type=jnp.float32)
    r = jax.nn.sigmoid(dot(m, wr_ref[...]) + dot(h, ur_ref[...]) + brz_ref[0:1, :])
    z = jax.nn.sigmoid(dot(m, wz_ref[...]) + dot(h, uz_ref[...]) + brz_ref[1:2, :])
    nn = jnp.tanh(dot(m, wn_ref[...]) + bn_i_ref[...] +
                  r * (dot(h, un_ref[...]) + bn_h_ref[...]))
    hn = (1.0 - z) * nn + z * h
    h_out[...] = hn
    msg_out[...] = dot(hn, wnext_ref[...])


def _gru_final_body(m_ref, h_ref, wr_ref, wz_ref, wn_ref, ur_ref, uz_ref, un_ref,
                    brz_ref, bn_i_ref, bn_h_ref, h_out):
    m = m_ref[...]
    h = h_ref[...]
    dot = lambda a, b: jnp.dot(a, b, preferred_element_type=jnp.float32)
    r = jax.nn.sigmoid(dot(m, wr_ref[...]) + dot(h, ur_ref[...]) + brz_ref[0:1, :])
    z = jax.nn.sigmoid(dot(m, wz_ref[...]) + dot(h, uz_ref[...]) + brz_ref[1:2, :])
    nn = jnp.tanh(dot(m, wn_ref[...]) + bn_i_ref[...] +
                  r * (dot(h, un_ref[...]) + bn_h_ref[...]))
    h_out[...] = jax.nn.relu((1.0 - z) * nn + z * h)


def _gru(m, h, gw, wnext):
    blk = lambda: pl.BlockSpec((NB, D), lambda i: (i, 0))
    mblk = lambda: pl.BlockSpec((NB, DP), lambda i: (i, 0))
    wspec = lambda: pl.BlockSpec((D, D), lambda i: (0, 0))
    mwspec = lambda: pl.BlockSpec((DP, D), lambda i: (0, 0))
    if wnext is None:
        return pl.pallas_call(
            _gru_final_body,
            grid=(NGRID,),
            in_specs=[mblk(), blk()] + [mwspec()] * 3 + [wspec()] * 3 +
                     [pl.BlockSpec((2, D), lambda i: (0, 0)),
                      pl.BlockSpec((1, D), lambda i: (0, 0)),
                      pl.BlockSpec((1, D), lambda i: (0, 0))],
            out_specs=blk(),
            out_shape=jax.ShapeDtypeStruct((N, D), jnp.float32),
        )(m, h, *gw)
    return pl.pallas_call(
        _gru_body,
        grid=(NGRID,),
        in_specs=[mblk(), blk()] + [mwspec()] * 3 + [wspec()] * 3 +
                 [pl.BlockSpec((2, D), lambda i: (0, 0)),
                  pl.BlockSpec((1, D), lambda i: (0, 0)),
                  pl.BlockSpec((1, D), lambda i: (0, 0)),
                  pl.BlockSpec((D, DP), lambda i: (0, 0))],
        out_specs=[blk(), mblk()],
        out_shape=[jax.ShapeDtypeStruct((N, D), jnp.float32),
                   jax.ShapeDtypeStruct((N, DP), jnp.float32)],
    )(m, h, *gw, wnext)


def _pool_body(b_ref, h_ref, s_ref, c_ref):
    pi = pl.program_id(0)

    @pl.when(pi == 0)
    def _():
        s_ref[...] = jnp.zeros_like(s_ref)
        c_ref[...] = jnp.zeros_like(c_ref)

    bcol = b_ref[0]
    oh = (bcol == lax.broadcasted_iota(jnp.int32, (NB, N_G), 1)).astype(jnp.float32)
    h = h_ref[...]
    s_ref[...] += lax.dot_general(oh, h, (((0,), (0,)), ((), ())),
                                  preferred_element_type=jnp.float32)
    c_ref[...] += lax.dot_general(oh, jnp.ones((NB, 8), jnp.float32),
                                  (((0,), (0,)), ((), ())),
                                  preferred_element_type=jnp.float32)


def _pool(h, batch3d):
    return pl.pallas_call(
        _pool_body,
        grid=(NGRID,),
        in_specs=[pl.BlockSpec((1, NB, 1), lambda i: (i, 0, 0)),
                  pl.BlockSpec((NB, D), lambda i: (i, 0))],
        out_specs=[pl.BlockSpec((N_G, D), lambda i: (0, 0)),
                   pl.BlockSpec((N_G, 8), lambda i: (0, 0))],
        out_shape=[jax.ShapeDtypeStruct((N_G, D), jnp.float32),
                   jax.ShapeDtypeStruct((N_G, 8), jnp.float32)],
    )(batch3d, h)


def _mlp_body(s1_ref, c1_ref, s2_ref, c2_ref, s3_ref, c3_ref,
              a1_ref, a2_ref, a3_ref, a4_ref, b1_ref, w2_ref, b2_ref,
              w3_ref, b3_ref, o_ref):
    h1 = s1_ref[...] / jnp.maximum(c1_ref[:, 0:1], 1.0)
    h2 = s2_ref[...] / jnp.maximum(c2_ref[:, 0:1], 1.0)
    h3 = s3_ref[...] / jnp.maximum(c3_ref[:, 0:1], 1.0)
    xx = h1 * h2 * h3
    dot = lambda a, b: jnp.dot(a, b, preferred_element_type=jnp.float32)
    y = jax.nn.relu(dot(h1, a1_ref[...]) + dot(h2, a2_ref[...]) +
                    dot(h3, a3_ref[...]) + dot(xx, a4_ref[...]) + b1_ref[...])
    y = jax.nn.relu(dot(y, w2_ref[...]) + b2_ref[...])
    o_ref[...] = dot(y, w3_ref[...]) + b3_ref[...]


def _branch_setup(x, edge_index, W, wih, whh, bih, bhh):
    srcs, dsts, offs = _prep_edges(edge_index)
    zpad = jnp.zeros((DP - D, D), jnp.float32)
    wr = jnp.concatenate([wih[0:D].T, zpad])
    wz = jnp.concatenate([wih[D:2 * D].T, zpad])
    wn = jnp.concatenate([wih[2 * D:].T, zpad])
    ur, uz, un = whh[0:D].T, whh[D:2 * D].T, whh[2 * D:].T
    brz = jnp.stack([bih[0:D] + bhh[0:D], bih[D:2 * D] + bhh[D:2 * D]])
    bn_i = bih[2 * D:].reshape(1, D)
    bn_h = bhh[2 * D:].reshape(1, D)
    gw = (wr, wz, wn, ur, uz, un, brz, bn_i, bn_h)
    L = W.shape[0]
    cpad = jnp.zeros((D, DP - D), jnp.float32)
    Wp = [jnp.concatenate([W[i], cpad], axis=1) for i in range(L)]
    return (srcs, dsts, offs), gw, Wp


def _branches(branch_args):
    setups = [_branch_setup(*a) for a in branch_args]
    L = branch_args[0][2].shape[0]
    hs = [a[0] for a in branch_args]
    msgs = [_mm(hs[k], setups[k][2][0]) for k in range(3)]
    for i in range(L):
        ms = [_segsum(msgs[k], *setups[k][0]) for k in range(3)]
        for k in range(3):
            if i + 1 < L:
                hs[k], msgs[k] = _gru(ms[k], hs[k], setups[k][1], setups[k][2][i + 1])
            else:
                hs[k] = _gru(ms[k], hs[k], setups[k][1], None)
    return hs


def kernel(x1, x2, x3, edge_index1, edge_index2, edge_index3, batch1, batch2, batch3,
           W1, wih1, whh1, bih1, bhh1, W2, wih2, whh2, bih2, bhh2, W3, wih3, whh3, bih3, bhh3,
           fc1_w, fc1_b, fc2_w, fc2_b, fc3_w, fc3_b):
    hf1, hf2, hf3 = _branches([
        (x1, edge_index1, W1, wih1, whh1, bih1, bhh1),
        (x2, edge_index2, W2, wih2, whh2, bih2, bhh2),
        (x3, edge_index3, W3, wih3, whh3, bih3, bhh3),
    ])

    s1, c1 = _pool(hf1, batch1.reshape(NGRID, NB, 1))
    s2, c2 = _pool(hf2, batch2.reshape(NGRID, NB, 1))
    s3, c3 = _pool(hf3, batch3.reshape(NGRID, NB, 1))

    fc_dim = fc1_w.shape[1]
    w1t = fc1_w.T
    a1, a2, a3, a4 = w1t[0:D], w1t[D:2 * D], w1t[2 * D:3 * D], w1t[3 * D:]
    b1 = fc1_b.reshape(1, -1)
    w2t = fc2_w.T
    b2 = fc2_b.reshape(1, -1)
    w3t = jnp.zeros((fc_dim, 8), fc3_w.dtype).at[:, :3].set(fc3_w.T)
    b3 = jnp.zeros((1, 8), fc3_b.dtype).at[0, :3].set(fc3_b)

    out = pl.pallas_call(
        _mlp_body,
        out_shape=jax.ShapeDtypeStruct((N_G, 8), jnp.float32),
    )(s1, c1, s2, c2, s3, c3, a1, a2, a3, a4, b1, w2t, b2, w3t, b3)
    return out[:, :3]
